# baseline (speedup 1.0000x reference)
import functools

import jax
import jax.numpy as jnp
from jax import lax
from jax.experimental import pallas as pl
from jax.experimental.pallas import tpu as pltpu
from jax.experimental.pallas import tpu_sc as plsc

N0 = 20000
NT = 16
B = 1280
NV = B // 16
NP = NT * B
G = 64
SG = 8
IMG_W = 1024.0
IMG_H = 1024.0
NEG_THRESH = 0.3
POS_THRESH = 0.7
N_FG = 128
N_BG = 128

_LN2 = 0.6931471805599453
_SQRT2 = 1.4142135623730951
_LNC = (6.43210144590789e-08, 1.0000040901688705, -0.5000199301348555,
        0.3329959787173782, -0.2488637832445017, 0.20655334597913605,
        -0.18852438788740203, 0.11589569104678855)


def _ln(x):
    b = plsc.bitcast(x, jnp.int32)
    e = lax.shift_right_arithmetic(b, 23) - 127
    m = plsc.bitcast((b & 0x007FFFFF) | 0x3F800000, jnp.float32)
    big = m > _SQRT2
    m = jnp.where(big, m * 0.5, m)
    e = jnp.where(big, e + 1, e)
    t = m - 1.0
    p = t * _LNC[7] + _LNC[6]
    for i in range(5, -1, -1):
        p = p * t + _LNC[i]
    return e.astype(jnp.float32) * _LN2 + p


_mesh = plsc.VectorSubcoreMesh(core_axis_name="c", subcore_axis_name="s",
                               num_cores=1)


@functools.partial(
    pl.kernel,
    mesh=_mesh,
    compiler_params=pltpu.CompilerParams(needs_layout_passes=False),
    out_type=[
        jax.ShapeDtypeStruct((NP * 4,), jnp.float32),
        jax.ShapeDtypeStruct((NP,), jnp.float32),
    ],
    scratch_types=[
        pltpu.VMEM((B,), jnp.float32),
        pltpu.VMEM((B,), jnp.float32),
        pltpu.VMEM((B,), jnp.float32),
        pltpu.VMEM((B,), jnp.float32),
        pltpu.VMEM((B,), jnp.float32),
        pltpu.VMEM((B,), jnp.float32),
        pltpu.VMEM((B,), jnp.float32),
        pltpu.VMEM((B,), jnp.int32),
        pltpu.VMEM((G * 16,), jnp.float32),
        pltpu.VMEM((G * 16,), jnp.int32),
        pltpu.VMEM((G,), jnp.float32),
        pltpu.VMEM((G,), jnp.float32),
        pltpu.VMEM((G,), jnp.float32),
        pltpu.VMEM((G,), jnp.float32),
        pltpu.VMEM((G,), jnp.float32),
        pltpu.VMEM((G,), jnp.float32),
        pltpu.VMEM((G,), jnp.float32),
        pltpu.VMEM((G,), jnp.float32),
        pltpu.VMEM((G,), jnp.float32),
        pltpu.VMEM((B * 4,), jnp.float32),
        pltpu.VMEM((G * 16,), jnp.float32),
        pltpu.VMEM((G * 16,), jnp.float32),
        pltpu.VMEM((G * 16,), jnp.float32),
        pltpu.VMEM((G * 16,), jnp.float32),
        pltpu.VMEM((G * 16,), jnp.float32),
        pltpu.VMEM((NT * G * 16,), jnp.float32),
        pltpu.VMEM((NT * G * 16,), jnp.int32),
        pltpu.VMEM((128,), jnp.float32),
        pltpu.VMEM((NT * 128,), jnp.float32),
        pltpu.VMEM((B,), jnp.float32),
        pltpu.VMEM((B,), jnp.float32),
        pltpu.VMEM((B,), jnp.int32),
        pltpu.VMEM((B,), jnp.int32),
        pltpu.VMEM((128,), jnp.int32),
        pltpu.VMEM((NT * 128,), jnp.int32),
        pltpu.VMEM_SHARED((NT * G * 16,), jnp.float32),
        pltpu.VMEM_SHARED((NT * G * 16,), jnp.int32),
        pltpu.VMEM_SHARED((NT * 128,), jnp.float32),
        pltpu.VMEM_SHARED((NT * 128,), jnp.int32),
    ],
)
def _sk(x1h, y1h, x2h, y2h, g1h, g2h, g3h, g4h,
        delta_h, lab_h,
        x1v, y1v, x2v, y2v, areav, insv, rmaxv, rargv, colmaxv, colargv,
        gc1, gc2, gc3, gc4, gareav, ggwv, gghv, ggcxv, ggcyv,
        deltav, spx1, spy1, spx2, spy2, spar, colall, argall,
        argstg, argloc, memberv, labv, poscumv, negcumv, cntv, cntall,
        colsh_max, colsh_arg, argsh, counts_sh):
    s = lax.axis_index("s")
    base = s * B
    lane = lax.broadcasted_iota(jnp.int32, (16,), 0)

    pltpu.sync_copy(x1h.at[pl.ds(base, B)], x1v)
    pltpu.sync_copy(y1h.at[pl.ds(base, B)], y1v)
    pltpu.sync_copy(x2h.at[pl.ds(base, B)], x2v)
    pltpu.sync_copy(y2h.at[pl.ds(base, B)], y2v)
    pltpu.sync_copy(g1h, gc1)
    pltpu.sync_copy(g2h, gc2)
    pltpu.sync_copy(g3h, gc3)
    pltpu.sync_copy(g4h, gc4)

    for k in range(G // 16):
        sl = pl.ds(k * 16, 16)
        a_ = gc1[sl]
        b_ = gc2[sl]
        c_ = gc3[sl]
        d_ = gc4[sl]
        gw = c_ - a_ + 1.0
        gh = d_ - b_ + 1.0
        ggwv[sl] = gw
        gghv[sl] = gh
        gareav[sl] = gw * gh
        ggcxv[sl] = a_ + 0.5 * gw
        ggcyv[sl] = b_ + 0.5 * gh

    def _pre(av, _):
        sl = pl.ds(av * 16, 16)
        ax1 = x1v[sl]
        ay1 = y1v[sl]
        ax2 = x2v[sl]
        ay2 = y2v[sl]
        areav[sl] = (ax2 - ax1 + 1.0) * (ay2 - ay1 + 1.0)
        ins = (ax1 >= 0.0) & (ay1 >= 0.0) & (ax2 <= IMG_W) & (ay2 <= IMG_H)
        insv[sl] = jnp.where(ins, 1.0, 0.0)
        rmaxv[sl] = jnp.full((16,), -3.4e38, jnp.float32)
        rargv[sl] = jnp.zeros((16,), jnp.int32)
        return 0

    lax.fori_loop(0, NV, _pre, 0)

    def _spl(g, _):
        gs = jnp.full((16,), g, jnp.int32)
        sl = pl.ds(g * 16, 16)
        spx1[sl] = plsc.load_gather(gc1, [gs])
        spy1[sl] = plsc.load_gather(gc2, [gs])
        spx2[sl] = plsc.load_gather(gc3, [gs])
        spy2[sl] = plsc.load_gather(gc4, [gs])
        spar[sl] = plsc.load_gather(gareav, [gs])
        return 0

    lax.fori_loop(0, G, _spl, 0)

    for strip in range(G // SG):
        g0 = strip * SG

        def _av(av, carry):
            cms = list(carry[:SG])
            cas = list(carry[SG:])
            sl = pl.ds(av * 16, 16)
            ax1 = x1v[sl]
            ay1 = y1v[sl]
            ax2 = x2v[sl]
            ay2 = y2v[sl]
            area = areav[sl]
            insb = insv[sl] > 0.0
            idxv = lane + (base + av * 16)
            rmax = rmaxv[sl]
            rarg = rargv[sl]
            for j in range(SG):
                g = g0 + j
                gsl = pl.ds(g * 16, 16)
                gs = jnp.full((16,), g, jnp.int32)
                gx1 = spx1[gsl]
                gy1 = spy1[gsl]
                gx2 = spx2[gsl]
                gy2 = spy2[gsl]
                gar = spar[gsl]
                iw = jnp.maximum(
                    jnp.minimum(ax2, gx2) - jnp.maximum(ax1, gx1) + 1.0, 0.0)
                ih = jnp.maximum(
                    jnp.minimum(ay2, gy2) - jnp.maximum(ay1, gy1) + 1.0, 0.0)
                inter = iw * ih
                union = area + gar - inter
                iou = inter / union
                rc = iou > rmax
                rmax = jnp.where(rc, iou, rmax)
                rarg = jnp.where(rc, gs, rarg)
                cc = (iou > cms[j]) & insb
                cms[j] = jnp.where(cc, iou, cms[j])
                cas[j] = jnp.where(cc, idxv, cas[j])
            rmaxv[sl] = rmax
            rargv[sl] = rarg
            return tuple(cms) + tuple(cas)

        init = tuple(jnp.full((16,), -1.0, jnp.float32) for _ in range(SG)) \
            + tuple(jnp.zeros((16,), jnp.int32) for _ in range(SG))
        fin = lax.fori_loop(0, NV, _av, init)
        for j in range(SG):
            colmaxv[pl.ds((g0 + j) * 16, 16)] = fin[j]
            colargv[pl.ds((g0 + j) * 16, 16)] = fin[SG + j]

    def _dl(av, _):
        sl = pl.ds(av * 16, 16)
        ax1 = x1v[sl]
        ay1 = y1v[sl]
        ax2 = x2v[sl]
        ay2 = y2v[sl]
        insb = insv[sl] > 0.0
        rmax = rmaxv[sl]
        rarg = rargv[sl]
        m_gw = plsc.load_gather(ggwv, [rarg])
        m_gh = plsc.load_gather(gghv, [rarg])
        m_gcx = plsc.load_gather(ggcxv, [rarg])
        m_gcy = plsc.load_gather(ggcyv, [rarg])
        ew = ax2 - ax1 + 1.0
        eh = ay2 - ay1 + 1.0
        ecx = ax1 + 0.5 * ew
        ecy = ay1 + 0.5 * eh
        dx = jnp.where(insb, (m_gcx - ecx) / ew, 0.0)
        dy = jnp.where(insb, (m_gcy - ecy) / eh, 0.0)
        dw = jnp.where(insb, _ln(m_gw / ew), 0.0)
        dh = jnp.where(insb, _ln(m_gh / eh), 0.0)
        li = (lane + av * 16) * 4
        plsc.store_scatter(deltav, [li], dx)
        plsc.store_scatter(deltav, [li + 1], dy)
        plsc.store_scatter(deltav, [li + 2], dw)
        plsc.store_scatter(deltav, [li + 3], dh)
        rmaxv[sl] = jnp.where(insb, rmax, -1.0)
        return 0

    lax.fori_loop(0, NV, _dl, 0)
    pltpu.sync_copy(deltav, delta_h.at[pl.ds(base * 4, B * 4)])

    pltpu.sync_copy(colmaxv, colsh_max.at[pl.ds(s * (G * 16), G * 16)])
    pltpu.sync_copy(colargv, colsh_arg.at[pl.ds(s * (G * 16), G * 16)])
    plsc.subcore_barrier()
    pltpu.sync_copy(colsh_max, colall)
    pltpu.sync_copy(colsh_arg, argall)

    ibig = jnp.full((16,), 0x7FFFFFFF, jnp.int32)
    pairs = jnp.zeros((16,), jnp.float32)
    for j in range(4):
        g = s * 4 + j
        goff = g * 16
        gm = colall[pl.ds(goff, 16)]
        for t in range(1, NT):
            gm = jnp.maximum(gm, colall[pl.ds(t * (G * 16) + goff, 16)])
        gmax = jnp.max(gm)
        gms = jnp.full((16,), gmax, jnp.float32)
        cand = ibig
        for t in range(NT):
            mv = colall[pl.ds(t * (G * 16) + goff, 16)]
            av_ = argall[pl.ds(t * (G * 16) + goff, 16)]
            cand = jnp.minimum(cand, jnp.where(mv == gms, av_, ibig))
        garg = jnp.min(cand)
        argf = plsc.bitcast(jnp.full((16,), garg, jnp.int32), jnp.float32)
        pairs = jnp.where(lane == 2 * j, gms, pairs)
        pairs = jnp.where(lane == 2 * j + 1, argf, pairs)
    for q in range(8):
        argstg[pl.ds(q * 16, 16)] = pairs if q == 0 else jnp.zeros(
            (16,), jnp.float32)
    pltpu.sync_copy(argstg, argsh.at[pl.ds(s * 128, 128)])
    plsc.subcore_barrier()
    pltpu.sync_copy(argsh, argloc)

    def _z(i, _):
        memberv[pl.ds(i * 16, 16)] = jnp.zeros((16,), jnp.float32)
        return 0

    lax.fori_loop(0, NV, _z, 0)

    ones = jnp.ones((16,), jnp.float32)
    for k in range(4):
        g = lane + k * 16
        idx1 = ((g >> 2) << 7) + ((g & 3) << 1) + 1
        garg = plsc.bitcast(plsc.load_gather(argloc, [idx1]), jnp.int32)
        msk = (garg >= base) & (garg < base + B)
        loc = jnp.clip(garg - base, 0, B - 1)
        plsc.store_scatter(memberv, [loc], ones, mask=msk)

    def _lab(i, carry):
        pc, nc = carry
        sl = pl.ds(i * 16, 16)
        rm = rmaxv[sl]
        mem = memberv[sl] > 0.0
        idxg = lane + (base + i * 16)
        lab0 = jnp.where(mem | (rm > POS_THRESH), 1.0,
                         jnp.where(rm < NEG_THRESH, 0.0, -1.0))
        validb = idxg < N0
        posi = jnp.where((lab0 == 1.0) & validb, 1, 0).astype(jnp.int32)
        negi = jnp.where((lab0 == 0.0) & validb, 1, 0).astype(jnp.int32)
        poscumv[sl] = plsc.cumsum(posi) + pc
        negcumv[sl] = plsc.cumsum(negi) + nc
        labv[sl] = lab0
        return pc + jnp.sum(posi), nc + jnp.sum(negi)

    pcnt, ncnt = lax.fori_loop(
        0, NV, _lab, (jnp.array(0, jnp.int32), jnp.array(0, jnp.int32)))
    cw = jnp.where(lane == 0, pcnt,
                   jnp.where(lane == 1, ncnt, 0)).astype(jnp.int32)
    for q in range(8):
        cntv[pl.ds(q * 16, 16)] = cw if q == 0 else jnp.zeros(
            (16,), jnp.int32)
    pltpu.sync_copy(cntv, counts_sh.at[pl.ds(s * 128, 128)])
    plsc.subcore_barrier()
    pltpu.sync_copy(counts_sh, cntall)
    pcol = plsc.load_gather(cntall, [lane * 128])
    ncol = plsc.load_gather(cntall, [lane * 128 + 1])
    pos_off = jnp.sum(jnp.where(lane < s, pcol, 0))
    neg_off = jnp.sum(jnp.where(lane < s, ncol, 0))

    def _fin(i, _):
        sl = pl.ds(i * 16, 16)
        lab = labv[sl]
        rm = rmaxv[sl]
        idxg = lane + (base + i * 16)
        validb = idxg < N0
        posb = (lab == 1.0) & validb
        negb = (lab == 0.0) & validb
        lab = jnp.where(posb & (poscumv[sl] + pos_off > N_FG), -1.0, lab)
        lab = jnp.where(negb & (negcumv[sl] + neg_off > N_BG), -1.0, lab)
        lab = jnp.where(rm < 0.0, -1.0, lab)
        labv[sl] = lab
        return 0

    lax.fori_loop(0, NV, _fin, 0)
    pltpu.sync_copy(labv, lab_h.at[pl.ds(base, B)])


def kernel(anchors, gt_bbox):
    pad = jnp.full((NP - N0, 4), 0.0, dtype=jnp.float32)
    pad = pad + jnp.array([-100.0, -100.0, -50.0, -50.0], dtype=jnp.float32)
    a = jnp.concatenate([anchors, pad], axis=0)
    x1, y1, x2, y2 = a[:, 0], a[:, 1], a[:, 2], a[:, 3]
    g1, g2, g3, g4 = gt_bbox[:, 0], gt_bbox[:, 1], gt_bbox[:, 2], gt_bbox[:, 3]
    delta_f, labels = _sk(x1, y1, x2, y2, g1, g2, g3, g4)
    delta = delta_f.reshape(NP, 4)[:N0]
    return delta, labels[:N0]


# speedup vs baseline: 1.3428x; 1.3428x over previous
import functools

import jax
import jax.numpy as jnp
from jax import lax
from jax.experimental import pallas as pl
from jax.experimental.pallas import tpu as pltpu
from jax.experimental.pallas import tpu_sc as plsc

N0 = 20000
NW = 32
B = 640
NV = B // 16
NP = NW * B
B2 = 1280
NV2 = B2 // 16
G = 64
IMG_W = 1024.0
IMG_H = 1024.0
NEG_THRESH = 0.3
POS_THRESH = 0.7
N_FG = 128
N_BG = 128

_LN2 = 0.6931471805599453
_SQRT2 = 1.4142135623730951
_LNC = (6.43210144590789e-08, 1.0000040901688705, -0.5000199301348555,
        0.3329959787173782, -0.2488637832445017, 0.20655334597913605,
        -0.18852438788740203, 0.11589569104678855)


def _ln(x):
    b = plsc.bitcast(x, jnp.int32)
    e = lax.shift_right_arithmetic(b, 23) - 127
    m = plsc.bitcast((b & 0x007FFFFF) | 0x3F800000, jnp.float32)
    big = m > _SQRT2
    m = jnp.where(big, m * 0.5, m)
    e = jnp.where(big, e + 1, e)
    t = m - 1.0
    p = t * _LNC[7] + _LNC[6]
    for i in range(5, -1, -1):
        p = p * t + _LNC[i]
    return e.astype(jnp.float32) * _LN2 + p


_mesh = plsc.VectorSubcoreMesh(core_axis_name="c", subcore_axis_name="s")


@functools.partial(
    pl.kernel,
    mesh=_mesh,
    compiler_params=pltpu.CompilerParams(needs_layout_passes=False),
    out_type=[
        jax.ShapeDtypeStruct((NP * 4,), jnp.float32),
        jax.ShapeDtypeStruct((NP,), jnp.float32),
        jax.ShapeDtypeStruct((512,), jnp.float32),
    ],
    scratch_types=[
        pltpu.VMEM((B,), jnp.float32),
        pltpu.VMEM((B,), jnp.float32),
        pltpu.VMEM((B,), jnp.float32),
        pltpu.VMEM((B,), jnp.float32),
        pltpu.VMEM((B,), jnp.float32),
        pltpu.VMEM((B,), jnp.float32),
        pltpu.VMEM((B,), jnp.float32),
        pltpu.VMEM((B,), jnp.int32),
        pltpu.VMEM((G * 16,), jnp.float32),
        pltpu.VMEM((G * 16,), jnp.int32),
        pltpu.VMEM((G,), jnp.float32),
        pltpu.VMEM((G,), jnp.float32),
        pltpu.VMEM((G,), jnp.float32),
        pltpu.VMEM((G,), jnp.float32),
        pltpu.VMEM((G,), jnp.float32),
        pltpu.VMEM((G,), jnp.float32),
        pltpu.VMEM((G,), jnp.float32),
        pltpu.VMEM((G,), jnp.float32),
        pltpu.VMEM((G,), jnp.float32),
        pltpu.VMEM((B * 4,), jnp.float32),
        pltpu.VMEM((G * 16,), jnp.float32),
        pltpu.VMEM((G * 16,), jnp.float32),
        pltpu.VMEM((G * 16,), jnp.float32),
        pltpu.VMEM((G * 16,), jnp.float32),
        pltpu.VMEM((G * 16,), jnp.float32),
        pltpu.VMEM((16 * G * 16,), jnp.float32),
        pltpu.VMEM((16 * G * 16,), jnp.int32),
        pltpu.VMEM((16,), jnp.float32),
        pltpu.VMEM_SHARED((16 * G * 16,), jnp.float32),
        pltpu.VMEM_SHARED((16 * G * 16,), jnp.int32),
    ],
)
def _k1(x1h, y1h, x2h, y2h, g1h, g2h, g3h, g4h,
        delta_h, rmax_h, colp_h,
        x1v, y1v, x2v, y2v, areav, insv, rmaxv, rargv, colmaxv, colargv,
        gc1, gc2, gc3, gc4, gareav, ggwv, gghv, ggcxv, ggcyv,
        deltav, spx1, spy1, spx2, spy2, spar,
        colall, argall, colpv, colsh_max, colsh_arg):
    c = lax.axis_index("c")
    s = lax.axis_index("s")
    wid = c * 16 + s
    base = wid * B
    lane = lax.broadcasted_iota(jnp.int32, (16,), 0)

    pltpu.sync_copy(x1h.at[pl.ds(base, B)], x1v)
    pltpu.sync_copy(y1h.at[pl.ds(base, B)], y1v)
    pltpu.sync_copy(x2h.at[pl.ds(base, B)], x2v)
    pltpu.sync_copy(y2h.at[pl.ds(base, B)], y2v)
    pltpu.sync_copy(g1h, gc1)
    pltpu.sync_copy(g2h, gc2)
    pltpu.sync_copy(g3h, gc3)
    pltpu.sync_copy(g4h, gc4)

    for k in range(G // 16):
        sl = pl.ds(k * 16, 16)
        a_ = gc1[sl]
        b_ = gc2[sl]
        c_ = gc3[sl]
        d_ = gc4[sl]
        gw = c_ - a_ + 1.0
        gh = d_ - b_ + 1.0
        ggwv[sl] = gw
        gghv[sl] = gh
        gareav[sl] = gw * gh
        ggcxv[sl] = a_ + 0.5 * gw
        ggcyv[sl] = b_ + 0.5 * gh

    def _pre(av, _):
        sl = pl.ds(av * 16, 16)
        ax1 = x1v[sl]
        ay1 = y1v[sl]
        ax2 = x2v[sl]
        ay2 = y2v[sl]
        areav[sl] = (ax2 - ax1 + 1.0) * (ay2 - ay1 + 1.0)
        ins = (ax1 >= 0.0) & (ay1 >= 0.0) & (ax2 <= IMG_W) & (ay2 <= IMG_H)
        insv[sl] = jnp.where(ins, 1.0, 0.0)
        rmaxv[sl] = jnp.full((16,), -3.4e38, jnp.float32)
        rargv[sl] = jnp.zeros((16,), jnp.int32)
        return 0

    plsc.parallel_loop(0, NV, unroll=2)(lambda av: _pre(av, 0))

    def _spl(g, _):
        gs = jnp.full((16,), g, jnp.int32)
        sl = pl.ds(g * 16, 16)
        spx1[sl] = plsc.load_gather(gc1, [gs])
        spy1[sl] = plsc.load_gather(gc2, [gs])
        spx2[sl] = plsc.load_gather(gc3, [gs])
        spy2[sl] = plsc.load_gather(gc4, [gs])
        spar[sl] = plsc.load_gather(gareav, [gs])
        return 0

    plsc.parallel_loop(0, G, unroll=2)(lambda g: _spl(g, 0))

    SG = 8
    for strip in range(G // SG):
        g0 = strip * SG

        def _av(av, carry):
            cms = list(carry[:SG])
            cas = list(carry[SG:])
            sl = pl.ds(av * 16, 16)
            ax1 = x1v[sl]
            ay1 = y1v[sl]
            ax2 = x2v[sl]
            ay2 = y2v[sl]
            area = areav[sl]
            insb = insv[sl] > 0.0
            idxv = lane + (base + av * 16)
            rmax = rmaxv[sl]
            rarg = rargv[sl]
            for j in range(SG):
                g = g0 + j
                gsl = pl.ds(g * 16, 16)
                gs = jnp.full((16,), g, jnp.int32)
                gx1 = spx1[gsl]
                gy1 = spy1[gsl]
                gx2 = spx2[gsl]
                gy2 = spy2[gsl]
                gar = spar[gsl]
                iw = jnp.maximum(
                    jnp.minimum(ax2, gx2) - jnp.maximum(ax1, gx1) + 1.0, 0.0)
                ih = jnp.maximum(
                    jnp.minimum(ay2, gy2) - jnp.maximum(ay1, gy1) + 1.0, 0.0)
                inter = iw * ih
                union = area + gar - inter
                iou = inter / union
                rc = iou > rmax
                rmax = jnp.where(rc, iou, rmax)
                rarg = jnp.where(rc, gs, rarg)
                cc = (iou > cms[j]) & insb
                cms[j] = jnp.where(cc, iou, cms[j])
                cas[j] = jnp.where(cc, idxv, cas[j])
            rmaxv[sl] = rmax
            rargv[sl] = rarg
            return tuple(cms) + tuple(cas)

        init = tuple(jnp.full((16,), -1.0, jnp.float32) for _ in range(SG)) \
            + tuple(jnp.zeros((16,), jnp.int32) for _ in range(SG))
        fin = plsc.parallel_loop(0, NV, carry=init, unroll=2)(_av)
        for j in range(SG):
            colmaxv[pl.ds((g0 + j) * 16, 16)] = fin[j]
            colargv[pl.ds((g0 + j) * 16, 16)] = fin[SG + j]

    def _dl(av, _):
        sl = pl.ds(av * 16, 16)
        ax1 = x1v[sl]
        ay1 = y1v[sl]
        ax2 = x2v[sl]
        ay2 = y2v[sl]
        insb = insv[sl] > 0.0
        rmax = rmaxv[sl]
        rarg = rargv[sl]
        m_gw = plsc.load_gather(ggwv, [rarg])
        m_gh = plsc.load_gather(gghv, [rarg])
        m_gcx = plsc.load_gather(ggcxv, [rarg])
        m_gcy = plsc.load_gather(ggcyv, [rarg])
        ew = ax2 - ax1 + 1.0
        eh = ay2 - ay1 + 1.0
        ecx = ax1 + 0.5 * ew
        ecy = ay1 + 0.5 * eh
        dx = jnp.where(insb, (m_gcx - ecx) / ew, 0.0)
        dy = jnp.where(insb, (m_gcy - ecy) / eh, 0.0)
        dw = jnp.where(insb, _ln(m_gw / ew), 0.0)
        dh = jnp.where(insb, _ln(m_gh / eh), 0.0)
        li = (lane + av * 16) * 4
        plsc.store_scatter(deltav, [li], dx)
        plsc.store_scatter(deltav, [li + 1], dy)
        plsc.store_scatter(deltav, [li + 2], dw)
        plsc.store_scatter(deltav, [li + 3], dh)
        rmaxv[sl] = jnp.where(insb, rmax, -1.0)
        return 0

    plsc.parallel_loop(0, NV, unroll=2)(lambda av: _dl(av, 0))

    pltpu.sync_copy(rmaxv, rmax_h.at[pl.ds(base, B)])
    pltpu.sync_copy(deltav, delta_h.at[pl.ds(base * 4, B * 4)])

    pltpu.sync_copy(colmaxv, colsh_max.at[pl.ds(s * (G * 16), G * 16)])
    pltpu.sync_copy(colargv, colsh_arg.at[pl.ds(s * (G * 16), G * 16)])
    plsc.subcore_barrier()
    pltpu.sync_copy(colsh_max, colall)
    pltpu.sync_copy(colsh_arg, argall)

    ibig = jnp.full((16,), 0x7FFFFFFF, jnp.int32)
    pairs = jnp.zeros((16,), jnp.float32)
    for j in range(4):
        g = s * 4 + j
        goff = g * 16
        gm = colall[pl.ds(goff, 16)]
        for t in range(1, 16):
            gm = jnp.maximum(gm, colall[pl.ds(t * (G * 16) + goff, 16)])
        gmax = jnp.max(gm)
        gms = jnp.full((16,), gmax, jnp.float32)
        cand = ibig
        for t in range(16):
            mv = colall[pl.ds(t * (G * 16) + goff, 16)]
            av_ = argall[pl.ds(t * (G * 16) + goff, 16)]
            cand = jnp.minimum(cand, jnp.where(mv == gms, av_, ibig))
        garg = jnp.min(cand)
        argf = plsc.bitcast(jnp.full((16,), garg, jnp.int32), jnp.float32)
        pairs = jnp.where(lane == 2 * j, gms, pairs)
        pairs = jnp.where(lane == 2 * j + 1, argf, pairs)
    colpv[...] = pairs
    pltpu.sync_copy(colpv, colp_h.at[pl.ds(c * 256 + s * 16, 16)])


@functools.partial(
    pl.kernel,
    mesh=_mesh,
    compiler_params=pltpu.CompilerParams(needs_layout_passes=False),
    out_type=jax.ShapeDtypeStruct((NP,), jnp.float32),
    scratch_types=[
        pltpu.VMEM((B2,), jnp.float32),
        pltpu.VMEM((B2,), jnp.float32),
        pltpu.VMEM((B2,), jnp.float32),
        pltpu.VMEM((B2,), jnp.int32),
        pltpu.VMEM((B2,), jnp.int32),
        pltpu.VMEM((512,), jnp.float32),
        pltpu.VMEM((128,), jnp.int32),
        pltpu.VMEM((2048,), jnp.int32),
        pltpu.VMEM_SHARED((2048,), jnp.int32),
    ],
)
def _k2(rmax_h, colp_h, lab_h,
        rmv, memberv, labv, poscumv, negcumv, colpv, cntv, cntall, counts_sh):
    c = lax.axis_index("c")
    s = lax.axis_index("s")
    lane = lax.broadcasted_iota(jnp.int32, (16,), 0)
    base2 = s * B2

    @pl.when(c == 0)
    def _work():
        pltpu.sync_copy(rmax_h.at[pl.ds(base2, B2)], rmv)
        pltpu.sync_copy(colp_h, colpv)

        def _z(i, _):
            memberv[pl.ds(i * 16, 16)] = jnp.zeros((16,), jnp.float32)
            return 0

        lax.fori_loop(0, NV2, _z, 0)

        ones = jnp.ones((16,), jnp.float32)
        for k in range(4):
            g = lane + k * 16
            idx0 = ((g >> 2) << 4) + ((g & 3) << 1)
            m0 = plsc.load_gather(colpv, [idx0])
            a0 = plsc.bitcast(plsc.load_gather(colpv, [idx0 + 1]), jnp.int32)
            m1 = plsc.load_gather(colpv, [idx0 + 256])
            a1 = plsc.bitcast(plsc.load_gather(colpv, [idx0 + 257]), jnp.int32)
            take1 = m1 > m0
            garg = jnp.where(take1, a1, a0)
            msk = (garg >= base2) & (garg < base2 + B2)
            loc = jnp.clip(garg - base2, 0, B2 - 1)
            plsc.store_scatter(memberv, [loc], ones, mask=msk)

        def _lab(i, carry):
            pc, nc = carry
            sl = pl.ds(i * 16, 16)
            rm = rmv[sl]
            mem = memberv[sl] > 0.0
            idxg = lane + (base2 + i * 16)
            lab0 = jnp.where(mem | (rm > POS_THRESH), 1.0,
                             jnp.where(rm < NEG_THRESH, 0.0, -1.0))
            validb = idxg < N0
            posi = jnp.where((lab0 == 1.0) & validb, 1, 0).astype(jnp.int32)
            negi = jnp.where((lab0 == 0.0) & validb, 1, 0).astype(jnp.int32)
            poscumv[sl] = plsc.cumsum(posi) + pc
            negcumv[sl] = plsc.cumsum(negi) + nc
            labv[sl] = lab0
            return pc + jnp.sum(posi), nc + jnp.sum(negi)

        pcnt, ncnt = lax.fori_loop(
            0, NV2, _lab,
            (jnp.array(0, jnp.int32), jnp.array(0, jnp.int32)))
        cw = jnp.where(lane == 0, pcnt,
                       jnp.where(lane == 1, ncnt, 0)).astype(jnp.int32)
        for q in range(8):
            cntv[pl.ds(q * 16, 16)] = cw if q == 0 else jnp.zeros(
                (16,), jnp.int32)
        pltpu.sync_copy(cntv, counts_sh.at[pl.ds(s * 128, 128)])

    plsc.subcore_barrier()

    @pl.when(c == 0)
    def _work2():
        pltpu.sync_copy(counts_sh, cntall)
        pcol = plsc.load_gather(cntall, [lane * 128])
        ncol = plsc.load_gather(cntall, [lane * 128 + 1])
        pos_off = jnp.sum(jnp.where(lane < s, pcol, 0))
        neg_off = jnp.sum(jnp.where(lane < s, ncol, 0))

        def _fin(i, _):
            sl = pl.ds(i * 16, 16)
            lab = labv[sl]
            rm = rmv[sl]
            idxg = lane + (base2 + i * 16)
            validb = idxg < N0
            posb = (lab == 1.0) & validb
            negb = (lab == 0.0) & validb
            lab = jnp.where(posb & (poscumv[sl] + pos_off > N_FG), -1.0, lab)
            lab = jnp.where(negb & (negcumv[sl] + neg_off > N_BG), -1.0, lab)
            lab = jnp.where(rm < 0.0, -1.0, lab)
            labv[sl] = lab
            return 0

        lax.fori_loop(0, NV2, _fin, 0)
        pltpu.sync_copy(labv, lab_h.at[pl.ds(base2, B2)])


def kernel(anchors, gt_bbox):
    pad = jnp.full((NP - N0, 4), 0.0, dtype=jnp.float32)
    pad = pad + jnp.array([-100.0, -100.0, -50.0, -50.0], dtype=jnp.float32)
    a = jnp.concatenate([anchors, pad], axis=0)
    x1, y1, x2, y2 = a[:, 0], a[:, 1], a[:, 2], a[:, 3]
    g1, g2, g3, g4 = gt_bbox[:, 0], gt_bbox[:, 1], gt_bbox[:, 2], gt_bbox[:, 3]
    delta_f, rmaxm, colp = _k1(x1, y1, x2, y2, g1, g2, g3, g4)
    labels = _k2(rmaxm, colp)
    delta = delta_f.reshape(NP, 4)[:N0]
    return delta, labels[:N0]
